# baseline (speedup 1.0000x reference)
import functools

import jax
import jax.numpy as jnp
from jax import lax
from jax.experimental import pallas as pl
from jax.experimental.pallas import tpu as pltpu
from jax.experimental.pallas import tpu_sc as plsc

N_NODES = 10000
D = 128

NC = 2
NS = 16
NW = NC * NS

CHUNK = 64
C0 = 60
C1 = 260
GCHUNKS = NS * (C0 + C1)
E_PAD = GCHUNKS * CHUNK

NBUF = 4
NIDX = 8

ROWS_PER_TILE = 632
N_PAD = ROWS_PER_TILE * NS

BM = 1000


def _mm_body(x_ref, deg_ref, w_ref, o_ref):
    o_ref[...] = jnp.dot(x_ref[...] * deg_ref[...], w_ref[...],
                         preferred_element_type=jnp.float32)


def _matmul(x, deg, w):
    grid = N_NODES // BM
    return pl.pallas_call(
        _mm_body,
        grid=(grid,),
        in_specs=[
            pl.BlockSpec((BM, D), lambda i: (i, 0)),
            pl.BlockSpec((BM, 1), lambda i: (i, 0)),
            pl.BlockSpec((D, D), lambda i: (0, 0)),
        ],
        out_specs=pl.BlockSpec((BM, D), lambda i: (i, 0)),
        out_shape=jax.ShapeDtypeStruct((N_NODES, D), jnp.float32),
    )(x, deg, w)


def _sc_aggregate(h, idx2, zeros):
    mesh = plsc.VectorSubcoreMesh(core_axis_name="c", subcore_axis_name="s")

    @functools.partial(
        pl.kernel,
        out_type=jax.ShapeDtypeStruct((NC, N_PAD, D), jnp.float32),
        mesh=mesh,
        scratch_types=[
            pltpu.VMEM_SHARED((N_PAD, D), jnp.float32),
            pltpu.VMEM((NIDX, 2, CHUNK), jnp.int32),
            pltpu.VMEM((NBUF, CHUNK, D), jnp.float32),
        ] + [pltpu.SemaphoreType.DMA] * (NBUF + NIDX),
    )
    def k(h_hbm, idx_hbm, zeros_hbm, out_hbm, acc, isl, rows, *sems):
        sem_g = sems[:NBUF]
        sem_i = sems[NBUF:]
        cid = lax.axis_index("c")
        sid = lax.axis_index("s")
        cpt = jnp.where(cid == 0, C0, C1)
        base = jnp.where(cid == 0, sid * C0, NS * C0 + sid * C1)
        groups = (cpt + NBUF + NIDX - 1) // NIDX

        pltpu.sync_copy(zeros_hbm, acc.at[pl.ds(sid * ROWS_PER_TILE, ROWS_PER_TILE)])
        plsc.subcore_barrier()

        for c in range(NBUF):
            pltpu.async_copy(idx_hbm.at[base + c], isl.at[c], sem_i[c])

        def body(g, _):
            for u in range(NIDX):
                s = g * NIDX + u
                b = u % NBUF
                jp = (u - NBUF) % NIDX

                @pl.when(jnp.logical_and(s >= NBUF, s < cpt + NBUF))
                def _retire():
                    pltpu.make_async_copy(h_hbm.at[isl.at[jp, 0]], rows.at[b],
                                          sem_g[b]).wait()
                    pltpu.sync_copy(rows.at[b], acc.at[isl.at[jp, 1]], add=True)

                @pl.when(s + NBUF < cpt)
                def _prefetch():
                    pltpu.async_copy(idx_hbm.at[base + s + NBUF], isl.at[jp],
                                     sem_i[jp])

                @pl.when(s < cpt)
                def _launch():
                    pltpu.make_async_copy(idx_hbm.at[base], isl.at[u],
                                          sem_i[u]).wait()
                    pltpu.async_copy(h_hbm.at[isl.at[u, 0]], rows.at[b], sem_g[b])

            return _

        lax.fori_loop(0, groups, body, None)

        plsc.subcore_barrier()
        pltpu.sync_copy(acc.at[pl.ds(sid * ROWS_PER_TILE, ROWS_PER_TILE)],
                        out_hbm.at[cid, pl.ds(sid * ROWS_PER_TILE, ROWS_PER_TILE)])

    return k(h, idx2, zeros)


def _comb_body(p_ref, deg_ref, b_ref, o_ref):
    o_ref[...] = (p_ref[0] + p_ref[1]) * deg_ref[...] + b_ref[...]


def _combine(partials, deg, bias):
    grid = N_NODES // BM
    return pl.pallas_call(
        _comb_body,
        grid=(grid,),
        in_specs=[
            pl.BlockSpec((NC, BM, D), lambda i: (0, i, 0)),
            pl.BlockSpec((BM, 1), lambda i: (i, 0)),
            pl.BlockSpec((1, D), lambda i: (0, 0)),
        ],
        out_specs=pl.BlockSpec((BM, D), lambda i: (i, 0)),
        out_shape=jax.ShapeDtypeStruct((N_NODES, D), jnp.float32),
    )(partials, deg, bias)


def kernel(x, edge_index, deg_inv_sqrt, weight, bias):
    src = edge_index[0].astype(jnp.int32)
    dst = edge_index[1].astype(jnp.int32)
    n_extra = E_PAD - src.shape[0]
    src = jnp.concatenate([src, jnp.zeros((n_extra,), jnp.int32)])
    dst = jnp.concatenate([dst, jnp.full((n_extra,), N_NODES, jnp.int32)])
    idx2 = jnp.stack(
        [src.reshape(GCHUNKS, CHUNK), dst.reshape(GCHUNKS, CHUNK)], axis=1)

    deg2d = deg_inv_sqrt[:, None]
    h = _matmul(x, deg2d, weight)
    zeros = jnp.zeros((ROWS_PER_TILE, D), jnp.float32)
    partials = _sc_aggregate(h, idx2, zeros)
    return _combine(partials, deg2d, bias.reshape(1, D))


# speedup vs baseline: 1.0840x; 1.0840x over previous
import functools

import jax
import jax.numpy as jnp
from jax import lax
from jax.experimental import pallas as pl
from jax.experimental.pallas import tpu as pltpu
from jax.experimental.pallas import tpu_sc as plsc

N_NODES = 10000
D = 128

NC = 2
NS = 16
NW = NC * NS

CHUNK = 64
C0 = 260
C1 = 60
GCHUNKS = NS * (C0 + C1)
E_PAD = GCHUNKS * CHUNK

NBUF = 4
NIDX = 8

ROWS_PER_TILE = 632
N_PAD = ROWS_PER_TILE * NS

BM = 1000


def _mm_body(x_ref, deg_ref, w_ref, o_ref):
    o_ref[...] = jnp.dot(x_ref[...] * deg_ref[...], w_ref[...],
                         preferred_element_type=jnp.float32)


def _matmul(x, deg, w):
    grid = N_NODES // BM
    return pl.pallas_call(
        _mm_body,
        grid=(grid,),
        in_specs=[
            pl.BlockSpec((BM, D), lambda i: (i, 0)),
            pl.BlockSpec((BM, 1), lambda i: (i, 0)),
            pl.BlockSpec((D, D), lambda i: (0, 0)),
        ],
        out_specs=pl.BlockSpec((BM, D), lambda i: (i, 0)),
        out_shape=jax.ShapeDtypeStruct((N_NODES, D), jnp.float32),
    )(x, deg, w)


def _sc_aggregate(h, idx2, zeros):
    mesh = plsc.VectorSubcoreMesh(core_axis_name="c", subcore_axis_name="s")

    @functools.partial(
        pl.kernel,
        out_type=jax.ShapeDtypeStruct((NC, N_PAD, D), jnp.float32),
        mesh=mesh,
        scratch_types=[
            pltpu.VMEM_SHARED((N_PAD, D), jnp.float32),
            pltpu.VMEM((NIDX, 2, CHUNK), jnp.int32),
            pltpu.VMEM((NBUF, CHUNK, D), jnp.float32),
        ] + [pltpu.SemaphoreType.DMA] * (NBUF + NIDX),
    )
    def k(h_hbm, idx_hbm, zeros_hbm, out_hbm, acc, isl, rows, *sems):
        sem_g = sems[:NBUF]
        sem_i = sems[NBUF:]
        cid = lax.axis_index("c")
        sid = lax.axis_index("s")
        cpt = jnp.where(cid == 0, C0, C1)
        base = jnp.where(cid == 0, sid * C0, NS * C0 + sid * C1)
        groups = (cpt + NBUF + NIDX - 1) // NIDX

        pltpu.sync_copy(zeros_hbm, acc.at[pl.ds(sid * ROWS_PER_TILE, ROWS_PER_TILE)])
        plsc.subcore_barrier()

        for c in range(NBUF):
            pltpu.async_copy(idx_hbm.at[base + c], isl.at[c], sem_i[c])

        def body(g, _):
            for u in range(NIDX):
                s = g * NIDX + u
                b = u % NBUF
                jp = (u - NBUF) % NIDX

                @pl.when(jnp.logical_and(s >= NBUF, s < cpt + NBUF))
                def _retire():
                    pltpu.make_async_copy(h_hbm.at[isl.at[jp, 0]], rows.at[b],
                                          sem_g[b]).wait()
                    pltpu.sync_copy(rows.at[b], acc.at[isl.at[jp, 1]], add=True)

                @pl.when(s + NBUF < cpt)
                def _prefetch():
                    pltpu.async_copy(idx_hbm.at[base + s + NBUF], isl.at[jp],
                                     sem_i[jp])

                @pl.when(s < cpt)
                def _launch():
                    pltpu.make_async_copy(idx_hbm.at[base], isl.at[u],
                                          sem_i[u]).wait()
                    pltpu.async_copy(h_hbm.at[isl.at[u, 0]], rows.at[b], sem_g[b])

            return _

        lax.fori_loop(0, groups, body, None)

        plsc.subcore_barrier()
        pltpu.sync_copy(acc.at[pl.ds(sid * ROWS_PER_TILE, ROWS_PER_TILE)],
                        out_hbm.at[cid, pl.ds(sid * ROWS_PER_TILE, ROWS_PER_TILE)])

    return k(h, idx2, zeros)


def _comb_body(p_ref, deg_ref, b_ref, o_ref):
    o_ref[...] = (p_ref[0] + p_ref[1]) * deg_ref[...] + b_ref[...]


def _combine(partials, deg, bias):
    grid = N_NODES // BM
    return pl.pallas_call(
        _comb_body,
        grid=(grid,),
        in_specs=[
            pl.BlockSpec((NC, BM, D), lambda i: (0, i, 0)),
            pl.BlockSpec((BM, 1), lambda i: (i, 0)),
            pl.BlockSpec((1, D), lambda i: (0, 0)),
        ],
        out_specs=pl.BlockSpec((BM, D), lambda i: (i, 0)),
        out_shape=jax.ShapeDtypeStruct((N_NODES, D), jnp.float32),
    )(partials, deg, bias)


def kernel(x, edge_index, deg_inv_sqrt, weight, bias):
    src = edge_index[0].astype(jnp.int32)
    dst = edge_index[1].astype(jnp.int32)
    n_extra = E_PAD - src.shape[0]
    src = jnp.concatenate([src, jnp.zeros((n_extra,), jnp.int32)])
    dst = jnp.concatenate([dst, jnp.full((n_extra,), N_NODES, jnp.int32)])
    idx2 = jnp.stack(
        [src.reshape(GCHUNKS, CHUNK), dst.reshape(GCHUNKS, CHUNK)], axis=1)

    deg2d = deg_inv_sqrt[:, None]
    h = _matmul(x, deg2d, weight)
    zeros = jnp.zeros((ROWS_PER_TILE, D), jnp.float32)
    partials = _sc_aggregate(h, idx2, zeros)
    return _combine(partials, deg2d, bias.reshape(1, D))
